# baseline (speedup 1.0000x reference)
import functools

import jax
import jax.numpy as jnp
from jax import lax
from jax.experimental import pallas as pl
from jax.experimental.pallas import tpu as pltpu
from jax.experimental.pallas import tpu_sc as plsc

LANES = 16
NSLOTS = 4


@functools.lru_cache(maxsize=None)
def _make_sc_kernel(n_rows, seq, hidden, n_types):
    info = plsc.get_sparse_core_info()
    nc, ns = info.num_cores, info.num_subcores
    nw = nc * ns
    assert n_rows % nw == 0
    rows_per_w = n_rows // nw
    assert seq % rows_per_w == 0
    CH = LANES
    n_chunks = rows_per_w // CH
    nh = hidden // LANES
    assert hidden % LANES == 0
    assert n_chunks % NSLOTS == 0 and n_chunks >= 2 * NSLOTS
    assert n_types == 2 * CH

    mesh = plsc.VectorSubcoreMesh(core_axis_name="c", subcore_axis_name="s")
    chunk_elems = CH * hidden

    def body(tt_hbm, seg_hbm, rowt_hbm, colt_hbm, pos_hbm, out_hbm,
             comb_v, p0, p1, p2, p3, o0, o1, o2, o3, tt_v,
             is0, is1, is2, is3, os0, os1, os2, os3):
        pbufs = (p0, p1, p2, p3)
        obufs = (o0, o1, o2, o3)
        in_sems = (is0, is1, is2, is3)
        out_sems = (os0, os1, os2, os3)

        cid = lax.axis_index("c")
        sid = lax.axis_index("s")
        wid = sid * nc + cid
        row_base = wid * rows_per_w
        s_base = lax.rem(row_base, seq)

        pltpu.sync_copy(seg_hbm, comb_v)
        pltpu.sync_copy(rowt_hbm.at[pl.ds(0, chunk_elems)], p0)
        pltpu.sync_copy(rowt_hbm.at[pl.ds(chunk_elems, chunk_elems)], p1)
        pltpu.sync_copy(colt_hbm.at[pl.ds(0, chunk_elems)], p2)
        pltpu.sync_copy(colt_hbm.at[pl.ds(chunk_elems, chunk_elems)], p3)
        pltpu.sync_copy(tt_hbm.at[pl.ds(row_base, rows_per_w)], tt_v)

        def combine_row(i, carry):
            base = i * hidden
            for j in range(nh):
                jo = j * LANES
                lo = pl.ds(base + jo, LANES)
                hi = pl.ds(base + CH * hidden + jo, LANES)
                bo = pl.ds(base + jo, LANES)
                comb_v[lo] = comb_v[lo] + p0[bo] + p2[bo]
                comb_v[hi] = comb_v[hi] + p1[bo] + p3[bo]
            return carry

        lax.fori_loop(0, CH, combine_row, 0)

        def in_copy(c, k):
            return pltpu.make_async_copy(
                pos_hbm.at[pl.ds((s_base + c * CH) * hidden, chunk_elems)],
                pbufs[k], in_sems[k])

        def out_copy(c, k):
            return pltpu.make_async_copy(
                obufs[k],
                out_hbm.at[pl.ds((row_base + c * CH) * hidden, chunk_elems)],
                out_sems[k])

        in_copy(0, 0).start()
        in_copy(1, 1).start()

        def step(g, carry):
            for k in range(NSLOTS):
                c = g * NSLOTS + k
                kp = (k + 2) % NSLOTS
                if k < 2:
                    in_copy(c + 2, kp).start()
                else:
                    @pl.when(g < (n_chunks // NSLOTS) - 1)
                    def _pre():
                        in_copy(c + 2, kp).start()
                @pl.when(g >= 1)
                def _wait_out():
                    out_copy(c - NSLOTS, k).wait()
                in_copy(c, k).wait()

                ttvec = tt_v[pl.ds(c * CH, CH)]
                tb = [ttvec[r] * hidden for r in range(CH)]
                pb = pbufs[k]
                ob = obufs[k]

                def jbody(j, carry2):
                    jo = j * LANES
                    for r in range(CH):
                        d = pl.ds(r * hidden + jo, LANES)
                        ob[d] = pb[d] + comb_v[pl.ds(tb[r] + jo, LANES)]
                    return carry2

                lax.fori_loop(0, nh, jbody, 0)
                out_copy(c, k).start()
            return carry

        lax.fori_loop(0, n_chunks // NSLOTS, step, 0)

        for k in range(NSLOTS):
            out_copy(n_chunks - NSLOTS + k, k).wait()

    return pl.kernel(
        body,
        out_type=jax.ShapeDtypeStruct((n_rows * hidden,), jnp.float32),
        mesh=mesh,
        scratch_types=(
            [pltpu.VMEM((n_types * hidden,), jnp.float32)]
            + [pltpu.VMEM((chunk_elems,), jnp.float32)] * (2 * NSLOTS)
            + [pltpu.VMEM((rows_per_w,), jnp.int32)]
            + [pltpu.SemaphoreType.DMA] * (2 * NSLOTS)
        ),
    )


def kernel(input_ids, token_type_ids, seg_table, pos_table, row_table,
           col_table):
    batch, seq = token_type_ids.shape
    n_types, hidden = seg_table.shape
    tt = token_type_ids.astype(jnp.int32).reshape(-1)
    sc = _make_sc_kernel(batch * seq, seq, hidden, n_types)
    out = sc(tt, seg_table.reshape(-1), row_table.reshape(-1),
             col_table.reshape(-1), pos_table.reshape(-1))
    return out.reshape(batch, seq, hidden)


# speedup vs baseline: 1.1137x; 1.1137x over previous
import functools

import jax
import jax.numpy as jnp
from jax import lax
from jax.experimental import pallas as pl
from jax.experimental.pallas import tpu as pltpu
from jax.experimental.pallas import tpu_sc as plsc

LANES = 16
NSLOTS = 4
JSUB = 8


@functools.lru_cache(maxsize=None)
def _make_sc_kernel(n_rows, seq, hidden, n_types):
    info = plsc.get_sparse_core_info()
    nc, ns = info.num_cores, info.num_subcores
    nw = nc * ns
    assert n_rows % nw == 0
    rows_per_w = n_rows // nw
    assert seq % rows_per_w == 0
    CH = LANES
    n_chunks = rows_per_w // CH
    nh = hidden // LANES
    assert hidden % LANES == 0 and nh % JSUB == 0
    assert n_chunks % NSLOTS == 0 and n_chunks >= 2 * NSLOTS
    assert n_types == 2 * CH

    mesh = plsc.VectorSubcoreMesh(core_axis_name="c", subcore_axis_name="s")
    chunk_elems = CH * hidden

    def body(tt_hbm, seg_hbm, rowt_hbm, colt_hbm, pos_hbm, out_hbm,
             comb_v, b0, b1, b2, b3, tt_v,
             is0, is1, is2, is3, os0, os1, os2, os3):
        bufs = (b0, b1, b2, b3)
        in_sems = (is0, is1, is2, is3)
        out_sems = (os0, os1, os2, os3)

        cid = lax.axis_index("c")
        sid = lax.axis_index("s")
        wid = sid * nc + cid
        row_base = wid * rows_per_w
        s_base = lax.rem(row_base, seq)

        pltpu.sync_copy(seg_hbm, comb_v)
        pltpu.sync_copy(rowt_hbm.at[pl.ds(0, chunk_elems)], b0)
        pltpu.sync_copy(rowt_hbm.at[pl.ds(chunk_elems, chunk_elems)], b1)
        pltpu.sync_copy(colt_hbm.at[pl.ds(0, chunk_elems)], b2)
        pltpu.sync_copy(colt_hbm.at[pl.ds(chunk_elems, chunk_elems)], b3)
        pltpu.sync_copy(tt_hbm.at[pl.ds(row_base, rows_per_w)], tt_v)

        def combine_row(i, carry):
            base = i * hidden
            for j in range(nh):
                jo = j * LANES
                lo = pl.ds(base + jo, LANES)
                hi = pl.ds(base + CH * hidden + jo, LANES)
                bo = pl.ds(base + jo, LANES)
                comb_v[lo] = comb_v[lo] + b0[bo] + b2[bo]
                comb_v[hi] = comb_v[hi] + b1[bo] + b3[bo]
            return carry

        lax.fori_loop(0, CH, combine_row, 0)

        def in_copy(c, k):
            return pltpu.make_async_copy(
                pos_hbm.at[pl.ds((s_base + c * CH) * hidden, chunk_elems)],
                bufs[k], in_sems[k])

        def out_copy(c, k):
            return pltpu.make_async_copy(
                bufs[k],
                out_hbm.at[pl.ds((row_base + c * CH) * hidden, chunk_elems)],
                out_sems[k])

        in_copy(0, 0).start()
        in_copy(1, 1).start()

        def step(g, carry):
            for k in range(NSLOTS):
                c = g * NSLOTS + k
                in_copy(c, k).wait()
                ttvec = tt_v[pl.ds(c * CH, CH)]
                tb = [ttvec[r] * hidden for r in range(CH)]
                buf = bufs[k]

                def jbody(jj, carry2):
                    jo = jj * (JSUB * LANES)
                    for r in range(CH):
                        br = tb[r] + jo
                        ro = r * hidden + jo
                        for u in range(JSUB):
                            plsc.addupdate(
                                buf.at[pl.ds(ro + u * LANES, LANES)],
                                comb_v[pl.ds(br + u * LANES, LANES)])
                    return carry2

                lax.fori_loop(0, nh // JSUB, jbody, 0)
                out_copy(c, k).start()

                kp = (k + 2) % NSLOTS
                if k < 2:
                    @pl.when(g >= 1)
                    def _wait():
                        out_copy(c + 2 - NSLOTS, kp).wait()
                    in_copy(c + 2, kp).start()
                else:
                    @pl.when(g < (n_chunks // NSLOTS) - 1)
                    def _pre():
                        out_copy(c + 2 - NSLOTS, kp).wait()
                        in_copy(c + 2, kp).start()
            return carry

        lax.fori_loop(0, n_chunks // NSLOTS, step, 0)

        for k in range(NSLOTS):
            out_copy(n_chunks - NSLOTS + k, k).wait()

    return pl.kernel(
        body,
        out_type=jax.ShapeDtypeStruct((n_rows * hidden,), jnp.float32),
        mesh=mesh,
        scratch_types=(
            [pltpu.VMEM((n_types * hidden,), jnp.float32)]
            + [pltpu.VMEM((chunk_elems,), jnp.float32)] * NSLOTS
            + [pltpu.VMEM((rows_per_w,), jnp.int32)]
            + [pltpu.SemaphoreType.DMA] * (2 * NSLOTS)
        ),
    )


def kernel(input_ids, token_type_ids, seg_table, pos_table, row_table,
           col_table):
    batch, seq = token_type_ids.shape
    n_types, hidden = seg_table.shape
    tt = token_type_ids.astype(jnp.int32).reshape(-1)
    sc = _make_sc_kernel(batch * seq, seq, hidden, n_types)
    out = sc(tt, seg_table.reshape(-1), row_table.reshape(-1),
             col_table.reshape(-1), pos_table.reshape(-1))
    return out.reshape(batch, seq, hidden)


# speedup vs baseline: 1.9481x; 1.7492x over previous
import functools

import jax
import jax.numpy as jnp
from jax import lax
from jax.experimental import pallas as pl
from jax.experimental.pallas import tpu as pltpu
from jax.experimental.pallas import tpu_sc as plsc

LANES = 16
NSLOTS = 4


@functools.lru_cache(maxsize=None)
def _make_sc_kernel(n_rows, seq, hidden, n_types):
    info = plsc.get_sparse_core_info()
    nc, ns = info.num_cores, info.num_subcores
    nw = nc * ns
    assert n_rows % nw == 0
    rows_per_w = n_rows // nw
    assert seq % rows_per_w == 0
    CH = LANES
    n_chunks = rows_per_w // CH
    nh = hidden // LANES
    assert hidden % LANES == 0
    assert n_chunks % NSLOTS == 0 and n_chunks >= 2 * NSLOTS
    assert n_types == 2 * CH

    mesh = plsc.VectorSubcoreMesh(core_axis_name="c", subcore_axis_name="s")
    chunk_elems = CH * hidden

    def body(tt_hbm, seg_hbm, rowt_hbm, colt_hbm, pos_hbm, out_hbm,
             comb_v, b0, b1, b2, b3, tt_v,
             is0, is1, is2, is3, os0, os1, os2, os3):
        bufs = (b0, b1, b2, b3)
        in_sems = (is0, is1, is2, is3)
        out_sems = (os0, os1, os2, os3)

        cid = lax.axis_index("c")
        sid = lax.axis_index("s")
        wid = sid * nc + cid
        row_base = wid * rows_per_w
        s_base = lax.rem(row_base, seq)

        pltpu.sync_copy(seg_hbm, comb_v)
        pltpu.sync_copy(rowt_hbm.at[pl.ds(0, CH)], b0)
        pltpu.sync_copy(rowt_hbm.at[pl.ds(CH, CH)], b1)
        pltpu.sync_copy(colt_hbm.at[pl.ds(0, CH)], b2)
        pltpu.sync_copy(colt_hbm.at[pl.ds(CH, CH)], b3)
        pltpu.sync_copy(tt_hbm.at[pl.ds(row_base, rows_per_w)], tt_v)

        def combine_row(i, carry):
            base = i * hidden
            hbase = (i + CH) * hidden
            for j in range(nh):
                jo = j * LANES
                jds = pl.ds(jo, LANES)
                lo = pl.ds(base + jo, LANES)
                hi = pl.ds(hbase + jo, LANES)
                comb_v[lo] = comb_v[lo] + b0[i, jds] + b2[i, jds]
                comb_v[hi] = comb_v[hi] + b1[i, jds] + b3[i, jds]
            return carry

        lax.fori_loop(0, CH, combine_row, 0)

        def in_copy(c, k):
            return pltpu.make_async_copy(
                pos_hbm.at[pl.ds(s_base + c * CH, CH)], bufs[k], in_sems[k])

        def out_copy(c, k):
            return pltpu.make_async_copy(
                bufs[k], out_hbm.at[pl.ds(row_base + c * CH, CH)],
                out_sems[k])

        in_copy(0, 0).start()
        in_copy(1, 1).start()

        def step(g, carry):
            for k in range(NSLOTS):
                c = g * NSLOTS + k
                in_copy(c, k).wait()
                ttvec = tt_v[pl.ds(c * CH, CH)]
                tb = [ttvec[r] * hidden for r in range(CH)]
                buf = bufs[k]

                def jbody(j, carry2):
                    jo = j * LANES
                    jds = pl.ds(jo, LANES)
                    for r in range(CH):
                        plsc.addupdate(buf.at[r, jds],
                                       comb_v[pl.ds(tb[r] + jo, LANES)])
                    return carry2

                lax.fori_loop(0, nh, jbody, 0)
                out_copy(c, k).start()

                kp = (k + 2) % NSLOTS
                if k < 2:
                    @pl.when(g >= 1)
                    def _wait():
                        out_copy(c + 2 - NSLOTS, kp).wait()
                    in_copy(c + 2, kp).start()
                else:
                    @pl.when(g < (n_chunks // NSLOTS) - 1)
                    def _pre():
                        out_copy(c + 2 - NSLOTS, kp).wait()
                        in_copy(c + 2, kp).start()
            return carry

        lax.fori_loop(0, n_chunks // NSLOTS, step, 0)

        for k in range(NSLOTS):
            out_copy(n_chunks - NSLOTS + k, k).wait()

    return pl.kernel(
        body,
        out_type=jax.ShapeDtypeStruct((n_rows, hidden), jnp.float32),
        mesh=mesh,
        scratch_types=(
            [pltpu.VMEM((n_types * hidden,), jnp.float32)]
            + [pltpu.VMEM((CH, hidden), jnp.float32)] * NSLOTS
            + [pltpu.VMEM((rows_per_w,), jnp.int32)]
            + [pltpu.SemaphoreType.DMA] * (2 * NSLOTS)
        ),
    )


def kernel(input_ids, token_type_ids, seg_table, pos_table, row_table,
           col_table):
    batch, seq = token_type_ids.shape
    n_types, hidden = seg_table.shape
    tt = token_type_ids.astype(jnp.int32).reshape(-1)
    sc = _make_sc_kernel(batch * seq, seq, hidden, n_types)
    out = sc(tt, seg_table.reshape(-1), row_table, col_table, pos_table)
    return out.reshape(batch, seq, hidden)


# speedup vs baseline: 3.2517x; 1.6692x over previous
import functools

import jax
import jax.numpy as jnp
from jax import lax
from jax.experimental import pallas as pl
from jax.experimental.pallas import tpu as pltpu
from jax.experimental.pallas import tpu_sc as plsc

LANES = 16
NSLOTS = 4


@functools.lru_cache(maxsize=None)
def _make_sc_kernel(n_rows, seq, hidden, n_types):
    info = plsc.get_sparse_core_info()
    nc, ns = info.num_cores, info.num_subcores
    nw = nc * ns
    assert n_rows % nw == 0
    rows_per_w = n_rows // nw
    assert seq % rows_per_w == 0
    CH = LANES
    n_chunks = rows_per_w // CH
    nh = hidden // LANES
    assert hidden % LANES == 0
    assert n_chunks % NSLOTS == 0 and n_chunks >= 2 * NSLOTS
    assert n_types == 2 * CH

    mesh = plsc.VectorSubcoreMesh(core_axis_name="c", subcore_axis_name="s")
    chunk_elems = CH * hidden

    def body(tt_hbm, seg_hbm, rowt_hbm, colt_hbm, pos_hbm, out_hbm,
             comb_v, b0, b1, b2, b3, tt_v,
             is0, is1, is2, is3, os0, os1, os2, os3):
        bufs = (b0, b1, b2, b3)
        in_sems = (is0, is1, is2, is3)
        out_sems = (os0, os1, os2, os3)

        cid = lax.axis_index("c")
        sid = lax.axis_index("s")
        wid = sid * nc + cid
        row_base = wid * rows_per_w
        s_base = lax.rem(row_base, seq)

        pltpu.sync_copy(seg_hbm, comb_v)
        pltpu.sync_copy(rowt_hbm.at[pl.ds(0, CH)], b0)
        pltpu.sync_copy(rowt_hbm.at[pl.ds(CH, CH)], b1)
        pltpu.sync_copy(colt_hbm.at[pl.ds(0, CH)], b2)
        pltpu.sync_copy(colt_hbm.at[pl.ds(CH, CH)], b3)
        pltpu.sync_copy(tt_hbm.at[pl.ds(row_base, rows_per_w)], tt_v)

        def combine_row(i, carry):
            base = i * hidden
            hbase = (i + CH) * hidden
            for j in range(nh):
                jo = j * LANES
                jds = pl.ds(jo, LANES)
                lo = pl.ds(base + jo, LANES)
                hi = pl.ds(hbase + jo, LANES)
                comb_v[lo] = comb_v[lo] + b0[i, jds] + b2[i, jds]
                comb_v[hi] = comb_v[hi] + b1[i, jds] + b3[i, jds]
            return carry

        lax.fori_loop(0, CH, combine_row, 0)

        def in_copy(c, k):
            return pltpu.make_async_copy(
                pos_hbm.at[pl.ds(s_base + c * CH, CH)], bufs[k], in_sems[k])

        def out_copy(c, k):
            return pltpu.make_async_copy(
                bufs[k], out_hbm.at[pl.ds(row_base + c * CH, CH)],
                out_sems[k])

        in_copy(0, 0).start()
        in_copy(1, 1).start()

        def step(g, carry):
            for k in range(NSLOTS):
                c = g * NSLOTS + k
                in_copy(c, k).wait()
                ttvec = tt_v[pl.ds(c * CH, CH)]
                tb = [ttvec[r] * hidden for r in range(CH)]
                buf = bufs[k]

                @plsc.parallel_loop(0, nh, unroll=2)
                def jbody(j):
                    jo = j * LANES
                    jds = pl.ds(jo, LANES)
                    for r in range(CH):
                        plsc.addupdate(buf.at[r, jds],
                                       comb_v[pl.ds(tb[r] + jo, LANES)])
                out_copy(c, k).start()

                kp = (k + 2) % NSLOTS
                if k < 2:
                    @pl.when(g >= 1)
                    def _wait():
                        out_copy(c + 2 - NSLOTS, kp).wait()
                    in_copy(c + 2, kp).start()
                else:
                    @pl.when(g < (n_chunks // NSLOTS) - 1)
                    def _pre():
                        out_copy(c + 2 - NSLOTS, kp).wait()
                        in_copy(c + 2, kp).start()
            return carry

        lax.fori_loop(0, n_chunks // NSLOTS, step, 0)

        for k in range(NSLOTS):
            out_copy(n_chunks - NSLOTS + k, k).wait()

    return pl.kernel(
        body,
        out_type=jax.ShapeDtypeStruct((n_rows, hidden), jnp.float32),
        mesh=mesh,
        scratch_types=(
            [pltpu.VMEM((n_types * hidden,), jnp.float32)]
            + [pltpu.VMEM((CH, hidden), jnp.float32)] * NSLOTS
            + [pltpu.VMEM((rows_per_w,), jnp.int32)]
            + [pltpu.SemaphoreType.DMA] * (2 * NSLOTS)
        ),
    )


def kernel(input_ids, token_type_ids, seg_table, pos_table, row_table,
           col_table):
    batch, seq = token_type_ids.shape
    n_types, hidden = seg_table.shape
    tt = token_type_ids.astype(jnp.int32).reshape(-1)
    sc = _make_sc_kernel(batch * seq, seq, hidden, n_types)
    out = sc(tt, seg_table.reshape(-1), row_table, col_table, pos_table)
    return out.reshape(batch, seq, hidden)
